# baseline (speedup 1.0000x reference)
import jax
import jax.numpy as jnp
from jax.experimental import pallas as pl


def _mlp_tail_body(pooled_ref, fW1_ref, fb1_ref, fW2_ref, fb2_ref, out_ref):
    h = jnp.maximum(pooled_ref[...], 0.0)
    h = jnp.maximum(h @ fW1_ref[...] + fb1_ref[...][None, :], 0.0)
    out_ref[...] = h @ fW2_ref[...] + fb2_ref[...][None, :]


def kernel(x, edge_index, edge_weight, batch, physics_score, W1, b1, g1, be1, W2, b2, g2, be2, W3, b3, g3, be3, W4, b4, g4, be4, W5, b5, g5, be5, fW1, fb1, fW2, fb2):
    n = x.shape[0]
    G = 64
    loop = jnp.arange(n)
    s = jnp.concatenate([edge_index[0], loop])
    d = jnp.concatenate([edge_index[1], loop])
    ew = jnp.concatenate([edge_weight, jnp.ones((n,), edge_weight.dtype)])
    deg = jnp.zeros((n,), jnp.float32).at[d].add(ew)
    dinv = jnp.where(deg > 0, 1.0 / jnp.sqrt(deg), 0.0)
    norm = dinv[s] * ew * dinv[d]

    def conv(h, W, b):
        h = h @ W
        out = jnp.zeros_like(h).at[d].add(h[s] * norm[:, None])
        return out + b

    def bn(h, g, be):
        return g * h * (1.0 / jnp.sqrt(1.0 + 1e-5)) + be

    h = bn(jax.nn.relu(conv(x, W1, b1)), g1, be1)
    h = bn(jax.nn.relu(conv(h, W2, b2)), g2, be2)
    h = bn(jax.nn.relu(conv(h, W3, b3)), g3, be3)
    h = jax.nn.relu(bn(conv(h, W4, b4), g4, be4))
    h = bn(conv(h, W5, b5), g5, be5)
    pooled = jax.ops.segment_sum(h, batch, num_segments=G)

    out = pl.pallas_call(
        _mlp_tail_body,
        out_shape=jax.ShapeDtypeStruct((G, 1), jnp.float32),
    )(pooled, fW1, fb1, fW2, fb2)
    return out.reshape(-1)


# speedup vs baseline: 6.9553x; 6.9553x over previous
import functools

import jax
import jax.numpy as jnp
from jax import lax
from jax.experimental import pallas as pl
from jax.experimental.pallas import tpu as pltpu
from jax.experimental.pallas import tpu_sc as plsc

_BN_C = 1.0 / (1.0 + 1e-5) ** 0.5


def _make_agg(E, N, dout, K=80, zs=200):
    info = plsc.get_sparse_core_info()
    NC, NS = info.num_cores, info.num_subcores
    NW = NC * NS
    e_per_w = E // NW
    n_chunks = e_per_w // K
    n_blocks = N // zs
    n_rounds = -(-n_blocks // NS)
    mesh = plsc.VectorSubcoreMesh(core_axis_name="c", subcore_axis_name="s")

    @functools.partial(
        pl.kernel,
        mesh=mesh,
        compiler_params=pltpu.CompilerParams(use_tc_tiling_on_sc=False),
        out_type=jax.ShapeDtypeStruct((NC, N, dout), jnp.float32),
        scratch_types=[
            pltpu.VMEM((K,), jnp.int32),
            pltpu.VMEM((K,), jnp.int32),
            pltpu.VMEM((K,), jnp.float32),
            pltpu.VMEM((K, dout), jnp.float32),
            pltpu.VMEM((zs, dout), jnp.float32),
            pltpu.VMEM_SHARED((N, dout), jnp.float32),
            pltpu.SemaphoreType.DMA,
        ],
    )
    def agg(hw_hbm, s_hbm, d_hbm, ew_hbm, out_hbm, sidx_v, didx_v, ew_v,
            rows_v, zslab, acc, sem):
        cid = lax.axis_index("c")
        sid = lax.axis_index("s")
        wid = sid * NC + cid

        zero16 = jnp.zeros((16,), jnp.float32)

        def zrow(i, _):
            for j in range(dout // 16):
                zslab[i, pl.ds(j * 16, 16)] = zero16
            return 0

        lax.fori_loop(0, zs, zrow, 0)
        for t in range(n_rounds):
            b = t * NS + sid

            @pl.when(b < n_blocks)
            def _():
                pltpu.sync_copy(zslab, acc.at[pl.ds(b * zs, zs)])

        plsc.subcore_barrier()

        def chunk(cix, _):
            base = wid * e_per_w + cix * K
            pltpu.sync_copy(s_hbm.at[pl.ds(base, K)], sidx_v)
            pltpu.sync_copy(d_hbm.at[pl.ds(base, K)], didx_v)
            pltpu.sync_copy(ew_hbm.at[pl.ds(base, K)], ew_v)
            pltpu.async_copy(hw_hbm.at[sidx_v], rows_v, sem).wait()

            def group(g, _):
                ewv = ew_v[pl.ds(g * 16, 16)]
                for e in range(16):
                    w = ewv[e]
                    r = g * 16 + e
                    for j in range(dout // 16):
                        rows_v[r, pl.ds(j * 16, 16)] = (
                            rows_v[r, pl.ds(j * 16, 16)] * w)
                return 0

            lax.fori_loop(0, K // 16, group, 0)
            pltpu.sync_copy(rows_v, acc.at[didx_v], add=True)
            return 0

        lax.fori_loop(0, n_chunks, chunk, 0)
        plsc.subcore_barrier()
        for t in range(n_rounds):
            b = t * NS + sid

            @pl.when(b < n_blocks)
            def _():
                pltpu.sync_copy(acc.at[pl.ds(b * zs, zs)],
                                out_hbm.at[cid].at[pl.ds(b * zs, zs)])

    return agg


_BLK = 1000


def _prep_body(degp_ref, x_ref, W_ref, hw_ref, dinv_ref):
    p = degp_ref[...]
    deg = 1.0 + p[0, :, 0:1] + p[1, :, 0:1]
    dinv = jnp.where(deg > 0, lax.rsqrt(deg), 0.0)
    dinv_ref[...] = dinv
    hw_ref[...] = dinv * jnp.dot(x_ref[...], W_ref[...],
                                 preferred_element_type=jnp.float32)


def _tc_prep(degp, x, W1, dout):
    N = x.shape[0]
    grid = N // _BLK
    return pl.pallas_call(
        _prep_body,
        grid=(grid,),
        in_specs=[
            pl.BlockSpec((2, _BLK, 16), lambda i: (0, i, 0)),
            pl.BlockSpec((_BLK, x.shape[1]), lambda i: (i, 0)),
            pl.BlockSpec(W1.shape, lambda i: (0, 0)),
        ],
        out_specs=[
            pl.BlockSpec((_BLK, dout), lambda i: (i, 0)),
            pl.BlockSpec((_BLK, 1), lambda i: (i, 0)),
        ],
        out_shape=[
            jax.ShapeDtypeStruct((N, dout), jnp.float32),
            jax.ShapeDtypeStruct((N, 1), jnp.float32),
        ],
    )(degp, x, W1)


def _post_body(mode, p_ref, hw_ref, dinv_ref, b_ref, g_ref, be_ref, W_ref,
               out_ref):
    p = p_ref[...]
    dinv = dinv_ref[...]
    z = dinv * (p[0] + p[1] + hw_ref[...]) + b_ref[...]
    if mode == "relu_bn":
        h = jnp.maximum(z, 0.0)
        h = g_ref[...] * h * _BN_C + be_ref[...]
    else:
        h = g_ref[...] * z * _BN_C + be_ref[...]
        h = jnp.maximum(h, 0.0)
    out_ref[...] = dinv * jnp.dot(h, W_ref[...],
                                  preferred_element_type=jnp.float32)


def _tc_post(mode, p, hw, dinv, b, g, be, Wn):
    N, dout = hw.shape
    dnext = Wn.shape[1]
    grid = N // _BLK
    return pl.pallas_call(
        functools.partial(_post_body, mode),
        grid=(grid,),
        in_specs=[
            pl.BlockSpec((2, _BLK, dout), lambda i: (0, i, 0)),
            pl.BlockSpec((_BLK, dout), lambda i: (i, 0)),
            pl.BlockSpec((_BLK, 1), lambda i: (i, 0)),
            pl.BlockSpec((1, dout), lambda i: (0, 0)),
            pl.BlockSpec((1, dout), lambda i: (0, 0)),
            pl.BlockSpec((1, dout), lambda i: (0, 0)),
            pl.BlockSpec(Wn.shape, lambda i: (0, 0)),
        ],
        out_specs=pl.BlockSpec((_BLK, dnext), lambda i: (i, 0)),
        out_shape=jax.ShapeDtypeStruct((N, dnext), jnp.float32),
    )(p, hw, dinv, b.reshape(1, -1), g.reshape(1, -1), be.reshape(1, -1), Wn)


def _head_body(p_ref, hw_ref, dinv_ref, b_ref, g_ref, be_ref, batch_ref,
               fW1_ref, fb1_ref, fW2_ref, fb2_ref, out_ref, pooled_acc):
    i = pl.program_id(0)
    nsteps = pl.num_programs(0)
    p = p_ref[...]
    dinv = dinv_ref[...]
    z = dinv * (p[0] + p[1] + hw_ref[...]) + b_ref[...]
    h = g_ref[...] * z * _BN_C + be_ref[...]
    gids = lax.broadcasted_iota(jnp.int32, (_BLK, 64), 1)
    m = (batch_ref[...] == gids).astype(jnp.float32)
    pb = lax.dot_general(m, h, (((0,), (0,)), ((), ())),
                         preferred_element_type=jnp.float32)

    @pl.when(i == 0)
    def _():
        pooled_acc[...] = jnp.zeros_like(pooled_acc)

    pooled_acc[...] += pb

    @pl.when(i == nsteps - 1)
    def _():
        po = jnp.maximum(pooled_acc[...], 0.0)
        t = jnp.maximum(jnp.dot(po, fW1_ref[...],
                                preferred_element_type=jnp.float32)
                        + fb1_ref[...], 0.0)
        out_ref[...] = jnp.dot(t, fW2_ref[...],
                               preferred_element_type=jnp.float32) + fb2_ref[...]


def _tc_head(p, hw, dinv, b, g, be, batch, fW1, fb1, fW2, fb2):
    N, dout = hw.shape
    grid = N // _BLK
    return pl.pallas_call(
        _head_body,
        grid=(grid,),
        in_specs=[
            pl.BlockSpec((2, _BLK, dout), lambda i: (0, i, 0)),
            pl.BlockSpec((_BLK, dout), lambda i: (i, 0)),
            pl.BlockSpec((_BLK, 1), lambda i: (i, 0)),
            pl.BlockSpec((1, dout), lambda i: (0, 0)),
            pl.BlockSpec((1, dout), lambda i: (0, 0)),
            pl.BlockSpec((1, dout), lambda i: (0, 0)),
            pl.BlockSpec((_BLK, 1), lambda i: (i, 0)),
            pl.BlockSpec(fW1.shape, lambda i: (0, 0)),
            pl.BlockSpec((1, fb1.shape[0]), lambda i: (0, 0)),
            pl.BlockSpec(fW2.shape, lambda i: (0, 0)),
            pl.BlockSpec((1, 1), lambda i: (0, 0)),
        ],
        out_specs=pl.BlockSpec((64, 1), lambda i: (0, 0)),
        out_shape=jax.ShapeDtypeStruct((64, 1), jnp.float32),
        scratch_shapes=[pltpu.VMEM((64, 128), jnp.float32)],
    )(p, hw, dinv, b.reshape(1, -1), g.reshape(1, -1), be.reshape(1, -1),
      batch.reshape(-1, 1), fW1, fb1.reshape(1, -1), fW2, fb2.reshape(1, -1))


def kernel(x, edge_index, edge_weight, batch, physics_score, W1, b1, g1, be1,
           W2, b2, g2, be2, W3, b3, g3, be3, W4, b4, g4, be4, W5, b5, g5, be5,
           fW1, fb1, fW2, fb2):
    N = x.shape[0]
    E = edge_index.shape[1]
    s = edge_index[0]
    d = edge_index[1]
    ew = edge_weight

    ones16 = jnp.ones((N, 16), jnp.float32)
    degp = _make_agg(E, N, 16)(ones16, s, d, ew)

    hw, dinv = _tc_prep(degp, x, W1, W1.shape[1])

    layer_params = [
        ("relu_bn", b1, g1, be1, W2),
        ("relu_bn", b2, g2, be2, W3),
        ("relu_bn", b3, g3, be3, W4),
        ("bn_relu", b4, g4, be4, W5),
    ]
    for mode, b, g, be, Wn in layer_params:
        p = _make_agg(E, N, hw.shape[1])(hw, s, d, ew)
        hw = _tc_post(mode, p, hw, dinv, b, g, be, Wn)

    p = _make_agg(E, N, hw.shape[1])(hw, s, d, ew)
    out = _tc_head(p, hw, dinv, b5, g5, be5, batch, fW1, fb1, fW2, fb2)
    return out.reshape(-1)


# speedup vs baseline: 17.5405x; 2.5219x over previous
import functools

import jax
import jax.numpy as jnp
from jax import lax
from jax.experimental import pallas as pl
from jax.experimental.pallas import tpu as pltpu
from jax.experimental.pallas import tpu_sc as plsc


def _make_agg(E, N, dout, use_norm, K=80, zs=200):
    info = plsc.get_sparse_core_info()
    NC, NS = info.num_cores, info.num_subcores
    NW = NC * NS
    e_per_w = E // NW
    nch = e_per_w // K
    n_pairs = (nch - 1) // 2 if nch % 2 == 1 else (nch - 2) // 2
    n_blocks = N // zs
    n_rounds = -(-n_blocks // NS)
    mesh = plsc.VectorSubcoreMesh(core_axis_name="c", subcore_axis_name="s")

    scratch = [
        pltpu.VMEM((2, K), jnp.int32),
        pltpu.VMEM((2, K), jnp.int32),
        pltpu.VMEM((K,), jnp.int32),
        pltpu.VMEM((K,), jnp.int32),
        pltpu.VMEM((K, dout), jnp.float32),
        pltpu.VMEM((K, dout), jnp.float32),
        pltpu.VMEM_SHARED((N, dout), jnp.float32),
    ]
    if use_norm:
        scratch.append(pltpu.VMEM((N,), jnp.float32))
    scratch.extend([pltpu.SemaphoreType.DMA] * 8)

    @functools.partial(
        pl.kernel,
        mesh=mesh,
        compiler_params=pltpu.CompilerParams(use_tc_tiling_on_sc=False,
                                             needs_layout_passes=False),
        out_type=jax.ShapeDtypeStruct((NC, N, dout), jnp.float32),
        scratch_types=scratch,
    )
    def agg(hw_hbm, sew_hbm, didx_hbm, zeros_hbm, *rest):
        if use_norm:
            (dinv_hbm, out_hbm, sew0, sew1, di0, di1, rows0, rows1,
             acc, dinv_v, si0, si1, sd0, sd1, g0, g1, s0, s1) = rest
        else:
            (out_hbm, sew0, sew1, di0, di1, rows0, rows1,
             acc, si0, si1, sd0, sd1, g0, g1, s0, s1) = rest
        sew = (sew0, sew1)
        di = (di0, di1)
        rows = (rows0, rows1)
        si = (si0, si1)
        sd = (sd0, sd1)
        sg = (g0, g1)
        ss = (s0, s1)
        cid = lax.axis_index("c")
        sid = lax.axis_index("s")
        wid = sid * NC + cid
        ch0 = wid * nch

        if use_norm:
            pltpu.sync_copy(dinv_hbm, dinv_v)

        for t in range(n_rounds):
            b = t * NS + sid

            @pl.when(b < n_blocks)
            def _():
                pltpu.sync_copy(zeros_hbm, acc.at[pl.ds(b * zs, zs)])

        plsc.subcore_barrier()

        def issue_sew(c, p):
            pltpu.async_copy(sew_hbm.at[ch0 + c], sew[p], si[p])

        def wait_sew(p):
            pltpu.make_async_copy(sew_hbm.at[ch0], sew[p], si[p]).wait()

        def issue_didx(c, p):
            pltpu.async_copy(didx_hbm.at[ch0 + c], di[p], sd[p])

        def wait_didx(p):
            pltpu.make_async_copy(didx_hbm.at[ch0], di[p], sd[p]).wait()

        def issue_gather(p):
            pltpu.async_copy(hw_hbm.at[sew[p].at[0]], rows[p], sg[p])

        def wait_gather(p):
            pltpu.make_async_copy(hw_hbm.at[sew[p].at[0]], rows[p],
                                  sg[p]).wait()

        def issue_scatter(p):
            pltpu.async_copy(rows[p], acc.at[di[p]], ss[p], add=True)

        def wait_scatter(p):
            pltpu.make_async_copy(rows[p], acc.at[di[p]], ss[p]).wait()

        def scale(p):
            def group(g, _):
                sl = pl.ds(g * 16, 16)
                wv = plsc.bitcast(sew[p][1, sl], jnp.float32)
                if use_norm:
                    dls = plsc.load_gather(dinv_v, [sew[p][0, sl]])
                    dld = plsc.load_gather(dinv_v, [di[p][sl]])
                    wv = (dls * wv) * dld
                for e in range(16):
                    w = wv[e]
                    r = g * 16 + e
                    for j in range(dout // 16):
                        rows[p][r, pl.ds(j * 16, 16)] = (
                            rows[p][r, pl.ds(j * 16, 16)] * w)
                return 0

            lax.fori_loop(0, K // 16, group, 0)

        issue_sew(0, 0)
        issue_sew(1, 1)
        issue_didx(0, 0)
        wait_sew(0)
        issue_gather(0)

        def step(c, par):
            wait_gather(par)

            @pl.when(c >= 1)
            def _():
                wait_scatter(1 - par)

            @pl.when(c + 1 < nch)
            def _():
                issue_didx(c + 1, 1 - par)
                wait_sew(1 - par)
                issue_gather(1 - par)

            scale(par)
            wait_didx(par)
            issue_scatter(par)

            @pl.when(c + 2 < nch)
            def _():
                issue_sew(c + 2, par)

        def pair(i, _):
            step(2 * i, 0)
            step(2 * i + 1, 1)
            return 0

        lax.fori_loop(0, n_pairs, pair, 0)
        if nch % 2 == 1:
            step(nch - 1, 0)
            wait_scatter(0)
        else:
            step(nch - 2, 0)
            step(nch - 1, 1)
            wait_scatter(1)

        plsc.subcore_barrier()
        for t in range(n_rounds):
            b = t * NS + sid

            @pl.when(b < n_blocks)
            def _():
                pltpu.sync_copy(acc.at[pl.ds(b * zs, zs)],
                                out_hbm.at[cid].at[pl.ds(b * zs, zs)])

    return agg


_BLK = 1000


def _bn(h, g, be):
    return g * h * (1.0 / jnp.sqrt(jnp.float32(1.0 + 1e-5))) + be


def _prep_body(degp_ref, x_ref, W_ref, hw_ref, dinv_ref):
    p = degp_ref[...]
    deg = 1.0 + p[0, :, 0:1] + p[1, :, 0:1]
    dinv = jnp.where(deg > 0, 1.0 / jnp.sqrt(deg), 0.0)
    dinv_ref[...] = dinv
    hw_ref[...] = jnp.dot(x_ref[...], W_ref[...],
                          preferred_element_type=jnp.float32)


def _tc_prep(degp, x, W1, dout):
    N = x.shape[0]
    grid = N // _BLK
    return pl.pallas_call(
        _prep_body,
        grid=(grid,),
        in_specs=[
            pl.BlockSpec((2, _BLK, 16), lambda i: (0, i, 0)),
            pl.BlockSpec((_BLK, x.shape[1]), lambda i: (i, 0)),
            pl.BlockSpec(W1.shape, lambda i: (0, 0)),
        ],
        out_specs=[
            pl.BlockSpec((_BLK, dout), lambda i: (i, 0)),
            pl.BlockSpec((_BLK, 1), lambda i: (i, 0)),
        ],
        out_shape=[
            jax.ShapeDtypeStruct((N, dout), jnp.float32),
            jax.ShapeDtypeStruct((N, 1), jnp.float32),
        ],
    )(degp, x, W1)


def _post_body(mode, p_ref, hw_ref, dinv_ref, b_ref, g_ref, be_ref, W_ref,
               out_ref):
    p = p_ref[...]
    dinv = dinv_ref[...]
    z = p[0] + p[1] + hw_ref[...] * (dinv * dinv) + b_ref[...]
    if mode == "relu_bn":
        h = _bn(jnp.maximum(z, 0.0), g_ref[...], be_ref[...])
    else:
        h = jnp.maximum(_bn(z, g_ref[...], be_ref[...]), 0.0)
    out_ref[...] = jnp.dot(h, W_ref[...], preferred_element_type=jnp.float32)


def _tc_post(mode, p, hw, dinv, b, g, be, Wn):
    N, dout = hw.shape
    dnext = Wn.shape[1]
    grid = N // _BLK
    return pl.pallas_call(
        functools.partial(_post_body, mode),
        grid=(grid,),
        in_specs=[
            pl.BlockSpec((2, _BLK, dout), lambda i: (0, i, 0)),
            pl.BlockSpec((_BLK, dout), lambda i: (i, 0)),
            pl.BlockSpec((_BLK, 1), lambda i: (i, 0)),
            pl.BlockSpec((1, dout), lambda i: (0, 0)),
            pl.BlockSpec((1, dout), lambda i: (0, 0)),
            pl.BlockSpec((1, dout), lambda i: (0, 0)),
            pl.BlockSpec(Wn.shape, lambda i: (0, 0)),
        ],
        out_specs=pl.BlockSpec((_BLK, dnext), lambda i: (i, 0)),
        out_shape=jax.ShapeDtypeStruct((N, dnext), jnp.float32),
    )(p, hw, dinv, b.reshape(1, -1), g.reshape(1, -1), be.reshape(1, -1), Wn)


def _head_body(p_ref, hw_ref, dinv_ref, b_ref, g_ref, be_ref, batch_ref,
               fW1_ref, fb1_ref, fW2_ref, fb2_ref, out_ref, pooled_acc):
    i = pl.program_id(0)
    nsteps = pl.num_programs(0)
    p = p_ref[...]
    dinv = dinv_ref[...]
    z = p[0] + p[1] + hw_ref[...] * (dinv * dinv) + b_ref[...]
    h = _bn(z, g_ref[...], be_ref[...])
    gids = lax.broadcasted_iota(jnp.int32, (_BLK, 64), 1)
    m = (batch_ref[...] == gids).astype(jnp.float32)
    pb = lax.dot_general(m, h, (((0,), (0,)), ((), ())),
                         preferred_element_type=jnp.float32,
                         precision=lax.Precision.HIGHEST)

    @pl.when(i == 0)
    def _():
        pooled_acc[...] = jnp.zeros_like(pooled_acc)

    pooled_acc[...] += pb

    @pl.when(i == nsteps - 1)
    def _():
        po = jnp.maximum(pooled_acc[...], 0.0)
        t = jnp.maximum(jnp.dot(po, fW1_ref[...],
                                preferred_element_type=jnp.float32)
                        + fb1_ref[...], 0.0)
        out_ref[...] = jnp.dot(t, fW2_ref[...],
                               preferred_element_type=jnp.float32) + fb2_ref[...]


def _tc_head(p, hw, dinv, b, g, be, batch, fW1, fb1, fW2, fb2):
    N, dout = hw.shape
    grid = N // _BLK
    return pl.pallas_call(
        _head_body,
        grid=(grid,),
        in_specs=[
            pl.BlockSpec((2, _BLK, dout), lambda i: (0, i, 0)),
            pl.BlockSpec((_BLK, dout), lambda i: (i, 0)),
            pl.BlockSpec((_BLK, 1), lambda i: (i, 0)),
            pl.BlockSpec((1, dout), lambda i: (0, 0)),
            pl.BlockSpec((1, dout), lambda i: (0, 0)),
            pl.BlockSpec((1, dout), lambda i: (0, 0)),
            pl.BlockSpec((_BLK, 1), lambda i: (i, 0)),
            pl.BlockSpec(fW1.shape, lambda i: (0, 0)),
            pl.BlockSpec((1, fb1.shape[0]), lambda i: (0, 0)),
            pl.BlockSpec(fW2.shape, lambda i: (0, 0)),
            pl.BlockSpec((1, 1), lambda i: (0, 0)),
        ],
        out_specs=pl.BlockSpec((64, 1), lambda i: (0, 0)),
        out_shape=jax.ShapeDtypeStruct((64, 1), jnp.float32),
        scratch_shapes=[pltpu.VMEM((64, 128), jnp.float32)],
    )(p, hw, dinv, b.reshape(1, -1), g.reshape(1, -1), be.reshape(1, -1),
      batch.reshape(-1, 1), fW1, fb1.reshape(1, -1), fW2, fb2.reshape(1, -1))


def kernel(x, edge_index, edge_weight, batch, physics_score, W1, b1, g1, be1,
           W2, b2, g2, be2, W3, b3, g3, be3, W4, b4, g4, be4, W5, b5, g5, be5,
           fW1, fb1, fW2, fb2):
    N = x.shape[0]
    E = edge_index.shape[1]
    K = 80
    nch_total = E // K
    s = edge_index[0]
    d = edge_index[1]
    ew = edge_weight

    sew = jnp.concatenate(
        [s.reshape(nch_total, 1, K),
         lax.bitcast_convert_type(ew, jnp.int32).reshape(nch_total, 1, K)],
        axis=1)
    didx = d.reshape(nch_total, K)

    ones16 = jnp.ones((N, 16), jnp.float32)
    z16 = jnp.zeros((200, 16), jnp.float32)
    degp = _make_agg(E, N, 16, use_norm=False)(ones16, sew, didx, z16)

    hw, dinv = _tc_prep(degp, x, W1, W1.shape[1])
    dinv_flat = dinv.reshape(-1)

    layer_params = [
        ("relu_bn", b1, g1, be1, W2),
        ("relu_bn", b2, g2, be2, W3),
        ("relu_bn", b3, g3, be3, W4),
        ("bn_relu", b4, g4, be4, W5),
    ]
    for mode, b, g, be, Wn in layer_params:
        p = _make_agg(E, N, hw.shape[1], use_norm=True)(
            hw, sew, didx, jnp.zeros((200, hw.shape[1]), jnp.float32),
            dinv_flat)
        hw = _tc_post(mode, p, hw, dinv, b, g, be, Wn)

    p = _make_agg(E, N, hw.shape[1], use_norm=True)(
        hw, sew, didx, jnp.zeros((200, hw.shape[1]), jnp.float32), dinv_flat)
    out = _tc_head(p, hw, dinv, b5, g5, be5, batch, fW1, fb1, fW2, fb2)
    return out.reshape(-1)
